# Initial kernel scaffold; baseline (speedup 1.0000x reference)
import jax
import jax.numpy as jnp
from jax.experimental import pallas as pl


def kernel(x, edge_index, W1l, b1l, W1r, W2l, b2l, W2r):
    raise NotImplementedError("write your pallas kernel here")


# speedup vs baseline: 4.5885x; 4.5885x over previous
import functools

import jax
import jax.numpy as jnp
from jax import lax
from jax.experimental import pallas as pl
from jax.experimental.pallas import tpu as pltpu
from jax.experimental.pallas import tpu_sc as plsc

_N = 10000
_NP = 10240
_E = 320000
_F = 128
_NC = 2
_NS = 16
_NW = _NC * _NS
_EPW = _E // _NW
_CH = 80
_NIT = _EPW // _CH
_RPS = _NP // _NS

_SC_MESH = plsc.VectorSubcoreMesh(core_axis_name="c", subcore_axis_name="s")


def _fill_zb(zb_v):
    zeros16 = jnp.zeros((16,), jnp.float32)
    for r in range(16):
        for c in range(_F // 16):
            zb_v[r, pl.ds(c * 16, 16)] = zeros16


def _zero_shared(sid, zb_v, sh):
    def zb_body(j, _):
        pltpu.sync_copy(zb_v, sh.at[pl.ds(sid * _RPS + j * 16, 16)])
        return _
    lax.fori_loop(0, _RPS // 16, zb_body, 0)


def _writeback(cid, sid, sh, bounce_v, out_hbm):
    def wb_body(j, _):
        ro = sid * _RPS + j * _CH
        pltpu.sync_copy(sh.at[pl.ds(ro, _CH)], bounce_v)
        pltpu.sync_copy(bounce_v, out_hbm.at[pl.ds(cid * _NP + ro, _CH)])
        return _
    lax.fori_loop(0, _RPS // _CH, wb_body, 0)


def _sc_agg_body(x_hbm, src_hbm, dst_hbm, out_hbm,
                 sh_acc, idx_v, dst_v, rows_v, zb_v, sem):
    cid = lax.axis_index("c")
    sid = lax.axis_index("s")
    wid = cid * _NS + sid
    _fill_zb(zb_v)
    _zero_shared(sid, zb_v, sh_acc)
    plsc.subcore_barrier()

    def it_body(i, _):
        base = wid * _EPW + i * _CH
        pltpu.sync_copy(src_hbm.at[pl.ds(base, _CH)], idx_v)
        pltpu.sync_copy(dst_hbm.at[pl.ds(base, _CH)], dst_v)
        pltpu.async_copy(x_hbm.at[idx_v], rows_v, sem).wait()
        pltpu.sync_copy(rows_v, sh_acc.at[dst_v], add=True)
        return _
    lax.fori_loop(0, _NIT, it_body, 0)
    plsc.subcore_barrier()
    _writeback(cid, sid, sh_acc, rows_v, out_hbm)


def _sc_cnt_body(dst_hbm, cnt_hbm, sh_cnt, dst_v, ones_v, zb_v):
    cid = lax.axis_index("c")
    sid = lax.axis_index("s")
    wid = cid * _NS + sid
    _fill_zb(zb_v)
    _zero_shared(sid, zb_v, sh_cnt)
    ones16 = jnp.ones((16,), jnp.float32)
    for r in range(_CH):
        for c in range(_F // 16):
            ones_v[r, pl.ds(c * 16, 16)] = ones16
    plsc.subcore_barrier()

    def it_body(i, _):
        base = wid * _EPW + i * _CH
        pltpu.sync_copy(dst_hbm.at[pl.ds(base, _CH)], dst_v)
        pltpu.sync_copy(ones_v, sh_cnt.at[dst_v], add=True)
        return _
    lax.fori_loop(0, _NIT, it_body, 0)
    plsc.subcore_barrier()
    _writeback(cid, sid, sh_cnt, ones_v, cnt_hbm)


_sc_agg = pl.kernel(
    _sc_agg_body,
    out_type=jax.ShapeDtypeStruct((_NC * _NP, _F), jnp.float32),
    mesh=_SC_MESH,
    scratch_types=[
        pltpu.VMEM_SHARED((_NP, _F), jnp.float32),
        pltpu.VMEM((_CH,), jnp.int32),
        pltpu.VMEM((_CH,), jnp.int32),
        pltpu.VMEM((_CH, _F), jnp.float32),
        pltpu.VMEM((16, _F), jnp.float32),
        pltpu.SemaphoreType.DMA,
    ],
)

_sc_cnt = pl.kernel(
    _sc_cnt_body,
    out_type=jax.ShapeDtypeStruct((_NC * _NP, _F), jnp.float32),
    mesh=_SC_MESH,
    scratch_types=[
        pltpu.VMEM_SHARED((_NP, _F), jnp.float32),
        pltpu.VMEM((_CH,), jnp.int32),
        pltpu.VMEM((_CH, _F), jnp.float32),
        pltpu.VMEM((16, _F), jnp.float32),
    ],
)

_BR = 512


def _tc1_body(p0, p1, c0, c1, x, w1l, w1r, b1, h_out):
    cnt = c0[:, 0:1] + c1[:, 0:1]
    rcp = 1.0 / jnp.maximum(cnt, 1.0)
    mean = (p0[...] + p1[...]) * rcp
    h = (jnp.dot(mean, w1l[...], preferred_element_type=jnp.float32)
         + jnp.dot(x[...], w1r[...], preferred_element_type=jnp.float32)
         + b1[...])
    h_out[...] = jnp.maximum(h, 0.0)


def _tc2_body(q0, q1, c0, c1, h, w2l, w2r, b2, o_out):
    cnt = c0[:, 0:1] + c1[:, 0:1]
    rcp = 1.0 / jnp.maximum(cnt, 1.0)
    mean = (q0[...] + q1[...]) * rcp
    o = (jnp.dot(mean, w2l[...], preferred_element_type=jnp.float32)
         + jnp.dot(h[...], w2r[...], preferred_element_type=jnp.float32)
         + b2[...])
    m = jnp.max(o, axis=-1, keepdims=True)
    lse = jnp.log(jnp.sum(jnp.exp(o - m), axis=-1, keepdims=True)) + m
    o_out[...] = o - lse


def _row_blocked(body):
    grid = (_NP // _BR,)
    rb = lambda i: (i, 0)
    full = lambda i: (0, 0)
    return pl.pallas_call(
        body,
        grid=grid,
        in_specs=[
            pl.BlockSpec((_BR, _F), rb),
            pl.BlockSpec((_BR, _F), rb),
            pl.BlockSpec((_BR, _F), rb),
            pl.BlockSpec((_BR, _F), rb),
            pl.BlockSpec((_BR, _F), rb),
            pl.BlockSpec((_F, _F), full),
            pl.BlockSpec((_F, _F), full),
            pl.BlockSpec((1, _F), full),
        ],
        out_specs=pl.BlockSpec((_BR, _F), rb),
        out_shape=jax.ShapeDtypeStruct((_NP, _F), jnp.float32),
    )


_tc1 = _row_blocked(_tc1_body)
_tc2 = _row_blocked(_tc2_body)


@jax.jit
def kernel(x, edge_index, W1l, b1l, W1r, W2l, b2l, W2r):
    xp = jnp.pad(x, ((0, _NP - _N), (0, 0)))
    src = edge_index[0]
    dst = edge_index[1]
    cnt = _sc_cnt(dst)
    p = _sc_agg(xp, src, dst)
    h = _tc1(p[:_NP], p[_NP:], cnt[:_NP], cnt[_NP:], xp,
             W1l.T, W1r.T, b1l[None, :])
    q = _sc_agg(h, src, dst)
    out = _tc2(q[:_NP], q[_NP:], cnt[:_NP], cnt[_NP:], h,
               W2l.T, W2r.T, b2l[None, :])
    return out[:_N]
